# baseline (speedup 1.0000x reference)
import functools

import jax
import jax.numpy as jnp
from jax import lax
from jax.experimental import pallas as pl
from jax.experimental.pallas import tpu as pltpu
from jax.experimental.pallas import tpu_sc as plsc

H = 64
NCELL = 256
NPAIR = NCELL * NCELL
NRELC = 3
TOTAL_BINS = NRELC * NPAIR
NWORK = 32
POS_PER_W = TOTAL_BINS // NWORK
LANES = 16
CHUNK = 4096

NE_AC = 50000
NE_BC = 100000
NE_MC = 5000
SHARD_ALIGN = NWORK * 128
TRASH = TOTAL_BINS
HIST_WORDS = TOTAL_BINS + 16
SLICE_PER_TILE = TOTAL_BINS // LANES
SENTINEL = 1 << 20


def _pad_ne(ne):
    return ((ne + SHARD_ALIGN - 1) // SHARD_ALIGN) * SHARD_ALIGN


SH_AC = _pad_ne(NE_AC) // NWORK
SH_BC = _pad_ne(NE_BC) // NWORK
SH_MC = _pad_ne(NE_MC) // NWORK


def _sc_hist_body(src_ac, dst_ac, src_bc, dst_bc, src_mc, dst_mc,
                  zeros_hbm, ones_hbm,
                  out_hbm,
                  sa_buf, da_buf, sb_buf, db_buf, sm_buf, dm_buf,
                  idx_ac, idx_bc, idx_mc, val_buf, hist_sp,
                  sem_a, sem_b, sem_m, sem_o, sem_s):
    cid = lax.axis_index("c")
    tid = lax.axis_index("s")
    wid = cid * (NWORK // 2) + tid

    def edge_copy(ref, buf, sh, sem):
        return pltpu.async_copy(ref.at[pl.ds(wid * sh, sh)], buf, sem)
    ca1 = edge_copy(src_ac, sa_buf, SH_AC, sem_a)
    ca2 = edge_copy(dst_ac, da_buf, SH_AC, sem_a)
    cb1 = edge_copy(src_bc, sb_buf, SH_BC, sem_b)
    cb2 = edge_copy(dst_bc, db_buf, SH_BC, sem_b)
    cm1 = edge_copy(src_mc, sm_buf, SH_MC, sem_m)
    cm2 = edge_copy(dst_mc, dm_buf, SH_MC, sem_m)
    co = pltpu.async_copy(ones_hbm, val_buf, sem_o)

    @pl.when(tid == 0)
    def _():
        pltpu.sync_copy(zeros_hbm, hist_sp)

    def build_idx(sbuf, dbuf, ibuf, sh, rel_off):
        def vb(i, c):
            s = sbuf[pl.ds(i * LANES, LANES)]
            d = dbuf[pl.ds(i * LANES, LANES)]
            ibuf[pl.ds(i * LANES, LANES)] = jnp.minimum(
                s * NCELL + d + rel_off, TRASH)
            return c
        lax.fori_loop(0, sh // LANES, vb, 0, unroll=4)

    ca1.wait(); ca2.wait()
    build_idx(sa_buf, da_buf, idx_ac, SH_AC, 0)
    cb1.wait(); cb2.wait()
    build_idx(sb_buf, db_buf, idx_bc, SH_BC, NPAIR)
    cm1.wait(); cm2.wait()
    build_idx(sm_buf, dm_buf, idx_mc, SH_MC, 2 * NPAIR)
    co.wait()
    plsc.subcore_barrier()

    s1 = pltpu.async_copy(val_buf.at[pl.ds(0, SH_AC)], hist_sp.at[idx_ac],
                          sem_s, add=True)
    s2 = pltpu.async_copy(val_buf.at[pl.ds(0, SH_BC)], hist_sp.at[idx_bc],
                          sem_s, add=True)
    s3 = pltpu.async_copy(val_buf.at[pl.ds(0, SH_MC)], hist_sp.at[idx_mc],
                          sem_s, add=True)
    s1.wait(); s2.wait(); s3.wait()

    plsc.subcore_barrier()

    pltpu.sync_copy(
        hist_sp.at[pl.ds(tid * SLICE_PER_TILE, SLICE_PER_TILE)],
        out_hbm.at[cid, pl.ds(tid * SLICE_PER_TILE, SLICE_PER_TILE)])


@jax.jit
def _sc_hist(ei_ac, ei_bc, ei_mc):
    def padded(ei, ne):
        np_ = _pad_ne(ne)
        src = jnp.concatenate(
            [ei[0], jnp.full((np_ - ne,), SENTINEL, jnp.int32)])
        dst = jnp.concatenate([ei[1], jnp.zeros((np_ - ne,), jnp.int32)])
        return src, dst

    sa, da = padded(ei_ac, NE_AC)
    sb, db = padded(ei_bc, NE_BC)
    sm, dm = padded(ei_mc, NE_MC)
    zeros_c = jnp.zeros((HIST_WORDS,), jnp.float32)
    ones_c = jnp.ones((SH_BC,), jnp.float32)
    mesh = plsc.VectorSubcoreMesh(core_axis_name="c", subcore_axis_name="s")
    return pl.kernel(
        _sc_hist_body,
        out_type=jax.ShapeDtypeStruct((2, TOTAL_BINS), jnp.float32),
        mesh=mesh,
        compiler_params=pltpu.CompilerParams(needs_layout_passes=False),
        scratch_types=[
            pltpu.VMEM((SH_AC,), jnp.int32),
            pltpu.VMEM((SH_AC,), jnp.int32),
            pltpu.VMEM((SH_BC,), jnp.int32),
            pltpu.VMEM((SH_BC,), jnp.int32),
            pltpu.VMEM((SH_MC,), jnp.int32),
            pltpu.VMEM((SH_MC,), jnp.int32),
            pltpu.VMEM((SH_AC,), jnp.int32),
            pltpu.VMEM((SH_BC,), jnp.int32),
            pltpu.VMEM((SH_MC,), jnp.int32),
            pltpu.VMEM((SH_BC,), jnp.float32),
            pltpu.VMEM_SHARED((HIST_WORDS,), jnp.float32),
            pltpu.SemaphoreType.DMA,
            pltpu.SemaphoreType.DMA,
            pltpu.SemaphoreType.DMA,
            pltpu.SemaphoreType.DMA,
            pltpu.SemaphoreType.DMA,
        ],
    )(sa, da, sb, db, sm, dm, zeros_c, ones_c)


def _tc_body(x_cellT, xa, xb, xm,
             Wla, bla, Wlb, blb, Wlm, blm,
             WfS0, WfDT0, bf0, WsS0, WsDT0, bs0,
             WfS1, WfDT1, bf1, WsS1, WsDT1, bs1,
             WfS2, WfDT2, bf2, WsS2, WsDT2, bs2,
             hist, WpT, bp, WoT, bo,
             out, accT, adT_s, bdT_s):
    f32 = jnp.float32
    dot = functools.partial(jnp.dot, preferred_element_type=f32)

    xct = x_cellT[...]
    accT[...] = 3.0 * xct

    xs_list = [
        dot(xa[...], Wla[...]) + bla[...],
        dot(xb[...], Wlb[...]) + blb[...],
        dot(xm[...], Wlm[...]) + blm[...],
    ]
    rel_params = [
        (WfS0, WfDT0, bf0, WsS0, WsDT0, bs0),
        (WfS1, WfDT1, bf1, WsS1, WsDT1, bs1),
        (WfS2, WfDT2, bf2, WsS2, WsDT2, bs2),
    ]

    clamp = lambda v: jnp.clip(v, -60.0, 60.0)
    for r in range(NRELC):
        WfS, WfDT, bf, WsS, WsDT, bs = rel_params[r]
        xs = xs_list[r]
        EAs = jnp.exp(-clamp(dot(xs, WfS[...])))
        EBs = jnp.exp(clamp(dot(xs, WsS[...])))
        adT_s[...] = jnp.exp(-clamp(dot(WfDT[...], xct) + bf[...]))
        bdT_s[...] = jnp.exp(clamp(dot(WsDT[...], xct) + bs[...]))
        C = hist[0, r] + hist[1, r]

        def hbody(h, carry, EAs=EAs, EBs=EBs, C=C):
            onehot = (lax.broadcasted_iota(jnp.int32, (H, 1), 0) == h)
            onehot = onehot.astype(f32)
            cea = dot(EAs, onehot)
            ceb = dot(EBs, onehot)
            rea = adT_s[pl.ds(h, 1), :]
            reb = bdT_s[pl.ds(h, 1), :]
            pq = jnp.log(1.0 + ceb * reb) / (1.0 + cea * rea)
            contrib = dot(jnp.ones((1, NCELL), f32), pq * C)
            accT[pl.ds(h, 1), :] += contrib
            return carry
        lax.fori_loop(0, H, hbody, 0, unroll=8)

    outT = jnp.maximum(accT[...], 0.0)
    hT = jax.nn.softplus(dot(WpT[...], outT) + bp[...])
    out[...] = dot(WoT[...], hT) + bo[...]


def _tc_call(*args):
    return pl.pallas_call(
        _tc_body,
        out_shape=jax.ShapeDtypeStruct((1, NCELL), jnp.float32),
        scratch_shapes=[pltpu.VMEM((H, NCELL), jnp.float32),
                        pltpu.VMEM((H, NCELL), jnp.float32),
                        pltpu.VMEM((H, NCELL), jnp.float32)],
    )(*args)


def kernel(x_atom, x_bond, x_motif, x_cell,
           ei_aa, ei_ab, ei_am, ei_bb, ei_bm, ei_mm, ei_ac, ei_bc, ei_mc,
           W_lina, b_lina, W_linb, b_linb, W_linm, b_linm, W_proj, b_proj,
           W_out, b_out,
           Wf_aa, bf_aa, Ws_aa, bs_aa,
           Wf_ab, bf_ab, Ws_ab, bs_ab,
           Wf_am, bf_am, Ws_am, bs_am,
           Wf_bb, bf_bb, Ws_bb, bs_bb,
           Wf_bm, bf_bm, Ws_bm, bs_bm,
           Wf_mm, bf_mm, Ws_mm, bs_mm,
           Wf_ac, bf_ac, Ws_ac, bs_ac,
           Wf_bc, bf_bc, Ws_bc, bs_bc,
           Wf_mc, bf_mc, Ws_mc, bs_mc):
    hist = _sc_hist(ei_ac, ei_bc, ei_mc).reshape(2, NRELC, NCELL, NCELL)

    def relp(Wf, bf, Ws, bs):
        return (Wf[H:], Wf[:H].T, bf.reshape(H, 1),
                Ws[H:], Ws[:H].T, bs.reshape(H, 1))

    yT = _tc_call(
        x_cell.T, x_atom[:NCELL], x_bond[:NCELL], x_motif[:NCELL],
        W_lina, b_lina.reshape(1, H),
        W_linb, b_linb.reshape(1, H),
        W_linm, b_linm.reshape(1, H),
        *relp(Wf_ac, bf_ac, Ws_ac, bs_ac),
        *relp(Wf_bc, bf_bc, Ws_bc, bs_bc),
        *relp(Wf_mc, bf_mc, Ws_mc, bs_mc),
        hist, W_proj.T, b_proj.reshape(H, 1),
        W_out.T, b_out.reshape(1, 1),
    )
    return yT.T


# speedup vs baseline: 1.4263x; 1.4263x over previous
import functools

import jax
import jax.numpy as jnp
from jax import lax
from jax.experimental import pallas as pl
from jax.experimental.pallas import tpu as pltpu
from jax.experimental.pallas import tpu_sc as plsc

H = 64
NCELL = 256
NPAIR = NCELL * NCELL
NRELC = 3
TOTAL_BINS = NRELC * NPAIR
NWORK = 32
POS_PER_W = TOTAL_BINS // NWORK
LANES = 16
CHUNK = 4096

NE_AC = 50000
NE_BC = 100000
NE_MC = 5000
SHARD_ALIGN = NWORK * 128
TRASH = TOTAL_BINS
HIST_WORDS = TOTAL_BINS + 16
SLICE_PER_TILE = TOTAL_BINS // LANES
SENTINEL = 1 << 20


def _pad_ne(ne):
    return ((ne + SHARD_ALIGN - 1) // SHARD_ALIGN) * SHARD_ALIGN


SH_AC = _pad_ne(NE_AC) // NWORK
SH_BC = _pad_ne(NE_BC) // NWORK
SH_MC = _pad_ne(NE_MC) // NWORK


def _sc_hist_body(src_ac, dst_ac, src_bc, dst_bc, src_mc, dst_mc,
                  zeros_hbm, ones_hbm,
                  out_hbm,
                  sa_buf, da_buf, sb_buf, db_buf, sm_buf, dm_buf,
                  idx_ac, idx_bc, idx_mc, val_buf, hist_sp,
                  sem_a, sem_b, sem_m, sem_o, sem_s):
    cid = lax.axis_index("c")
    tid = lax.axis_index("s")
    wid = cid * (NWORK // 2) + tid

    def edge_copy(ref, buf, sh, sem):
        return pltpu.async_copy(ref.at[pl.ds(wid * sh, sh)], buf, sem)
    ca1 = edge_copy(src_ac, sa_buf, SH_AC, sem_a)
    ca2 = edge_copy(dst_ac, da_buf, SH_AC, sem_a)
    cb1 = edge_copy(src_bc, sb_buf, SH_BC, sem_b)
    cb2 = edge_copy(dst_bc, db_buf, SH_BC, sem_b)
    cm1 = edge_copy(src_mc, sm_buf, SH_MC, sem_m)
    cm2 = edge_copy(dst_mc, dm_buf, SH_MC, sem_m)
    co = pltpu.async_copy(ones_hbm, val_buf, sem_o)

    @pl.when(tid == 0)
    def _():
        pltpu.sync_copy(zeros_hbm, hist_sp)

    def build_idx(sbuf, dbuf, ibuf, sh, rel_off):
        def vb(i, c):
            s = sbuf[pl.ds(i * LANES, LANES)]
            d = dbuf[pl.ds(i * LANES, LANES)]
            ibuf[pl.ds(i * LANES, LANES)] = jnp.minimum(
                s * NCELL + d + rel_off, TRASH)
            return c
        lax.fori_loop(0, sh // LANES, vb, 0, unroll=4)

    ca1.wait(); ca2.wait()
    build_idx(sa_buf, da_buf, idx_ac, SH_AC, 0)
    cb1.wait(); cb2.wait()
    build_idx(sb_buf, db_buf, idx_bc, SH_BC, NPAIR)
    cm1.wait(); cm2.wait()
    build_idx(sm_buf, dm_buf, idx_mc, SH_MC, 2 * NPAIR)
    co.wait()
    plsc.subcore_barrier()

    s1 = pltpu.async_copy(val_buf.at[pl.ds(0, SH_AC)], hist_sp.at[idx_ac],
                          sem_s, add=True)
    s2 = pltpu.async_copy(val_buf.at[pl.ds(0, SH_BC)], hist_sp.at[idx_bc],
                          sem_s, add=True)
    s3 = pltpu.async_copy(val_buf.at[pl.ds(0, SH_MC)], hist_sp.at[idx_mc],
                          sem_s, add=True)
    s1.wait(); s2.wait(); s3.wait()

    plsc.subcore_barrier()

    pltpu.sync_copy(
        hist_sp.at[pl.ds(tid * SLICE_PER_TILE, SLICE_PER_TILE)],
        out_hbm.at[cid, pl.ds(tid * SLICE_PER_TILE, SLICE_PER_TILE)])


@jax.jit
def _sc_hist(ei_ac, ei_bc, ei_mc):
    def padded(ei, ne):
        np_ = _pad_ne(ne)
        src = jnp.concatenate(
            [ei[0], jnp.full((np_ - ne,), SENTINEL, jnp.int32)])
        dst = jnp.concatenate([ei[1], jnp.zeros((np_ - ne,), jnp.int32)])
        return src, dst

    sa, da = padded(ei_ac, NE_AC)
    sb, db = padded(ei_bc, NE_BC)
    sm, dm = padded(ei_mc, NE_MC)
    zeros_c = jnp.zeros((HIST_WORDS,), jnp.float32)
    ones_c = jnp.ones((SH_BC,), jnp.float32)
    mesh = plsc.VectorSubcoreMesh(core_axis_name="c", subcore_axis_name="s")
    return pl.kernel(
        _sc_hist_body,
        out_type=jax.ShapeDtypeStruct((2, TOTAL_BINS), jnp.float32),
        mesh=mesh,
        compiler_params=pltpu.CompilerParams(needs_layout_passes=False),
        scratch_types=[
            pltpu.VMEM((SH_AC,), jnp.int32),
            pltpu.VMEM((SH_AC,), jnp.int32),
            pltpu.VMEM((SH_BC,), jnp.int32),
            pltpu.VMEM((SH_BC,), jnp.int32),
            pltpu.VMEM((SH_MC,), jnp.int32),
            pltpu.VMEM((SH_MC,), jnp.int32),
            pltpu.VMEM((SH_AC,), jnp.int32),
            pltpu.VMEM((SH_BC,), jnp.int32),
            pltpu.VMEM((SH_MC,), jnp.int32),
            pltpu.VMEM((SH_BC,), jnp.float32),
            pltpu.VMEM_SHARED((HIST_WORDS,), jnp.float32),
            pltpu.SemaphoreType.DMA,
            pltpu.SemaphoreType.DMA,
            pltpu.SemaphoreType.DMA,
            pltpu.SemaphoreType.DMA,
            pltpu.SemaphoreType.DMA,
        ],
    )(sa, da, sb, db, sm, dm, zeros_c, ones_c)


def _tc_body(x_cellT, xa, xb, xm,
             Wla, bla, Wlb, blb, Wlm, blm,
             WfS0, WfDT0, bf0, WsS0, WsDT0, bs0,
             WfS1, WfDT1, bf1, WsS1, WsDT1, bs1,
             WfS2, WfDT2, bf2, WsS2, WsDT2, bs2,
             hist, WpT, bp, WoT, bo,
             out, accT, adT_s, bdT_s):
    f32 = jnp.float32
    dot = functools.partial(jnp.dot, preferred_element_type=f32)

    xct = x_cellT[...]
    accT[...] = 3.0 * xct

    xs_list = [
        dot(xa[...], Wla[...]) + bla[...],
        dot(xb[...], Wlb[...]) + blb[...],
        dot(xm[...], Wlm[...]) + blm[...],
    ]
    rel_params = [
        (WfS0, WfDT0, bf0, WsS0, WsDT0, bs0),
        (WfS1, WfDT1, bf1, WsS1, WsDT1, bs1),
        (WfS2, WfDT2, bf2, WsS2, WsDT2, bs2),
    ]

    clamp = lambda v: jnp.clip(v, -60.0, 60.0)
    eas_parts, ebs_parts = [], []
    for r in range(NRELC):
        WfS, WfDT, bf, WsS, WsDT, bs = rel_params[r]
        xs = xs_list[r]
        eas_parts.append(jnp.exp(-clamp(dot(xs, WfS[...]))))
        ebs_parts.append(jnp.exp(clamp(dot(xs, WsS[...]))))
        adT_s[pl.ds(r * H, H), :] = jnp.exp(
            -clamp(dot(WfDT[...], xct) + bf[...]))
        bdT_s[pl.ds(r * H, H), :] = jnp.exp(
            clamp(dot(WsDT[...], xct) + bs[...]))
    EAs = jnp.concatenate(eas_parts, axis=0)
    EBs = jnp.concatenate(ebs_parts, axis=0)
    C = (hist[0] + hist[1]).reshape(NRELC * NCELL, NCELL)

    def hbody(h, carry):
        onehot = (lax.broadcasted_iota(jnp.int32, (H, 1), 0) == h)
        onehot = onehot.astype(f32)
        cea = dot(EAs, onehot)
        ceb = dot(EBs, onehot)
        rea = jnp.concatenate(
            [jnp.broadcast_to(adT_s[pl.ds(h + r * H, 1), :], (NCELL, NCELL))
             for r in range(NRELC)], axis=0)
        reb = jnp.concatenate(
            [jnp.broadcast_to(bdT_s[pl.ds(h + r * H, 1), :], (NCELL, NCELL))
             for r in range(NRELC)], axis=0)
        pq = jnp.log(1.0 + ceb * reb) / (1.0 + cea * rea)
        contrib = jnp.sum(pq * C, axis=0, keepdims=True)
        accT[pl.ds(h, 1), :] += contrib
        return carry
    lax.fori_loop(0, H, hbody, 0, unroll=4)

    outT = jnp.maximum(accT[...], 0.0)
    hT = jax.nn.softplus(dot(WpT[...], outT) + bp[...])
    out[...] = dot(WoT[...], hT) + bo[...]


def _tc_call(*args):
    return pl.pallas_call(
        _tc_body,
        out_shape=jax.ShapeDtypeStruct((1, NCELL), jnp.float32),
        scratch_shapes=[pltpu.VMEM((H, NCELL), jnp.float32),
                        pltpu.VMEM((NRELC * H, NCELL), jnp.float32),
                        pltpu.VMEM((NRELC * H, NCELL), jnp.float32)],
    )(*args)


def kernel(x_atom, x_bond, x_motif, x_cell,
           ei_aa, ei_ab, ei_am, ei_bb, ei_bm, ei_mm, ei_ac, ei_bc, ei_mc,
           W_lina, b_lina, W_linb, b_linb, W_linm, b_linm, W_proj, b_proj,
           W_out, b_out,
           Wf_aa, bf_aa, Ws_aa, bs_aa,
           Wf_ab, bf_ab, Ws_ab, bs_ab,
           Wf_am, bf_am, Ws_am, bs_am,
           Wf_bb, bf_bb, Ws_bb, bs_bb,
           Wf_bm, bf_bm, Ws_bm, bs_bm,
           Wf_mm, bf_mm, Ws_mm, bs_mm,
           Wf_ac, bf_ac, Ws_ac, bs_ac,
           Wf_bc, bf_bc, Ws_bc, bs_bc,
           Wf_mc, bf_mc, Ws_mc, bs_mc):
    hist = _sc_hist(ei_ac, ei_bc, ei_mc).reshape(2, NRELC, NCELL, NCELL)

    def relp(Wf, bf, Ws, bs):
        return (Wf[H:], Wf[:H].T, bf.reshape(H, 1),
                Ws[H:], Ws[:H].T, bs.reshape(H, 1))

    yT = _tc_call(
        x_cell.T, x_atom[:NCELL], x_bond[:NCELL], x_motif[:NCELL],
        W_lina, b_lina.reshape(1, H),
        W_linb, b_linb.reshape(1, H),
        W_linm, b_linm.reshape(1, H),
        *relp(Wf_ac, bf_ac, Ws_ac, bs_ac),
        *relp(Wf_bc, bf_bc, Ws_bc, bs_bc),
        *relp(Wf_mc, bf_mc, Ws_mc, bs_mc),
        hist, W_proj.T, b_proj.reshape(H, 1),
        W_out.T, b_out.reshape(1, 1),
    )
    return yT.T
